# baseline (speedup 1.0000x reference)
import functools

import jax
import jax.numpy as jnp
from jax import lax
from jax.experimental import pallas as pl
from jax.experimental.pallas import tpu as pltpu
from jax.experimental.pallas import tpu_sc as plsc

_N = 10000
_E = 320000
_D = 128
_LAT = 64

_NC = 2
_NS = 16
_L = 16
_NW = _NC * _NS

_NP = 10240
_EP = 327680
_EPT = _EP // _NW
_KCH = _EPT // 128
_RPT = _NP // _NS

_SELU_ALPHA = 1.6732632423543772
_SELU_SCALE = 1.0507009873554805


def _selu(v):
    return _SELU_SCALE * jnp.where(
        v > 0, v, _SELU_ALPHA * (jnp.exp(v) - 1.0))


def _tc1_body(x_ref, wn_ref, ws_ref, b_ref, y_ref, xs_ref):
    x = x_ref[...]
    y_ref[...] = jnp.dot(x, wn_ref[...], preferred_element_type=jnp.float32)
    xs_ref[...] = (
        jnp.dot(x, ws_ref[...], preferred_element_type=jnp.float32)
        + b_ref[...])


def _tc1(xp, W_n1, W_s1, b1):
    return pl.pallas_call(
        _tc1_body,
        out_shape=(
            jax.ShapeDtypeStruct((_NP, _D), jnp.float32),
            jax.ShapeDtypeStruct((_NP, _D), jnp.float32),
        ),
    )(xp, W_n1, W_s1, b1)


def _tc2_body(xs_ref, agg_ref, cnt_ref, wn_ref, ws_ref, b_ref,
              y2_ref, hs_ref):
    agg = agg_ref[0] + agg_ref[1]
    invc = 1.0 / jnp.maximum(cnt_ref[...][:, 0:1], 1.0)
    h1 = jax.nn.relu(xs_ref[...] + agg * invc)
    y2_ref[...] = jnp.dot(h1, wn_ref[...], preferred_element_type=jnp.float32)
    hs_ref[...] = (
        jnp.dot(h1, ws_ref[...], preferred_element_type=jnp.float32)
        + b_ref[...])


def _tc2(xs, aggp, cntp, W_n2, W_s2, b2):
    R = 2048
    grid = _NP // R
    return pl.pallas_call(
        _tc2_body,
        grid=(grid,),
        in_specs=[
            pl.BlockSpec((R, _D), lambda i: (i, 0)),
            pl.BlockSpec((_NC, R, _D), lambda i: (0, i, 0)),
            pl.BlockSpec((R, 128), lambda i: (i, 0)),
            pl.BlockSpec((_D, _D), lambda i: (0, 0)),
            pl.BlockSpec((_D, _D), lambda i: (0, 0)),
            pl.BlockSpec((1, _D), lambda i: (0, 0)),
        ],
        out_specs=(
            pl.BlockSpec((R, _D), lambda i: (i, 0)),
            pl.BlockSpec((R, _D), lambda i: (i, 0)),
        ),
        out_shape=(
            jax.ShapeDtypeStruct((_NP, _D), jnp.float32),
            jax.ShapeDtypeStruct((_NP, _D), jnp.float32),
        ),
    )(xs, aggp, cntp, W_n2, W_s2, b2)


def _ln(v, g, b):
    m = jnp.mean(v, axis=-1, keepdims=True)
    var = jnp.mean((v - m) ** 2, axis=-1, keepdims=True)
    return (v - m) / jnp.sqrt(var + 1e-5) * g + b


def _tc3_body(hs_ref, agg_ref, cnt_ref, eps_ref,
              wm1_ref, bm1_ref, gm_ref, btm_ref, wm2_ref, bm2_ref,
              wm3_ref, bm3_ref,
              wl1_ref, bl1_ref, gl_ref, btl_ref, wl2_ref, bl2_ref,
              wl3_ref, bl3_ref,
              wdt_ref, bd1_ref, wdb_ref,
              ab_ref):
    agg = agg_ref[0] + agg_ref[1]
    invc = 1.0 / jnp.maximum(cnt_ref[...][:, 0:1], 1.0)
    h = hs_ref[...] + agg * invc

    m = _selu(_ln(jnp.dot(h, wm1_ref[...],
                          preferred_element_type=jnp.float32) + bm1_ref[...],
                  gm_ref[...], btm_ref[...]))
    m = _selu(jnp.dot(m, wm2_ref[...],
                      preferred_element_type=jnp.float32) + bm2_ref[...])
    mu = jnp.dot(m, wm3_ref[...],
                 preferred_element_type=jnp.float32) + bm3_ref[...]

    l = _selu(_ln(jnp.dot(h, wl1_ref[...],
                          preferred_element_type=jnp.float32) + bl1_ref[...],
                  gl_ref[...], btl_ref[...]))
    l = _selu(jnp.dot(l, wl2_ref[...],
                      preferred_element_type=jnp.float32) + bl2_ref[...])
    logstd = jnp.minimum(
        jnp.dot(l, wl3_ref[...],
                preferred_element_type=jnp.float32) + bl3_ref[...], 10.0)

    z = mu + eps_ref[...] * jnp.exp(logstd)
    a = jnp.dot(z, wdt_ref[...],
                preferred_element_type=jnp.float32) + bd1_ref[...]
    b = jnp.dot(z, wdb_ref[...],
                preferred_element_type=jnp.float32)
    ab_ref[...] = jnp.concatenate([a, b], axis=-1)


def _tc3(hs, aggp2, cntp, epsp, Wm1, bm1, gm, btm, Wm2, bm2, Wm3, bm3,
         Wl1, bl1, gl, btl, Wl2, bl2, Wl3, bl3, Wd1_top, bd1, Wd1_bot):
    R = 2048
    grid = _NP // R
    full = lambda shape: pl.BlockSpec(shape, lambda i: tuple(0 for _ in shape))
    return pl.pallas_call(
        _tc3_body,
        grid=(grid,),
        in_specs=[
            pl.BlockSpec((R, _D), lambda i: (i, 0)),
            pl.BlockSpec((_NC, R, _D), lambda i: (0, i, 0)),
            pl.BlockSpec((R, 128), lambda i: (i, 0)),
            pl.BlockSpec((R, _LAT), lambda i: (i, 0)),
            full((_D, 2 * _LAT)), full((1, 2 * _LAT)),
            full((1, 2 * _LAT)), full((1, 2 * _LAT)),
            full((2 * _LAT, 256)), full((1, 256)),
            full((256, _LAT)), full((1, _LAT)),
            full((_D, 2 * _LAT)), full((1, 2 * _LAT)),
            full((1, 2 * _LAT)), full((1, 2 * _LAT)),
            full((2 * _LAT, 256)), full((1, 256)),
            full((256, _LAT)), full((1, _LAT)),
            full((_LAT, _LAT)), full((1, _LAT)), full((_LAT, _LAT)),
        ],
        out_specs=pl.BlockSpec((R, _D), lambda i: (i, 0)),
        out_shape=jax.ShapeDtypeStruct((_NP, _D), jnp.float32),
    )(hs, aggp2, cntp, epsp, Wm1, bm1, gm, btm, Wm2, bm2, Wm3, bm3,
      Wl1, bl1, gl, btl, Wl2, bl2, Wl3, bl3, Wd1_top, bd1, Wd1_bot)


_MESH = plsc.VectorSubcoreMesh(core_axis_name="c", subcore_axis_name="s")


def _unpack_indices(pk_v, src_v, dst_v, nchunks):
    def chunk(k, carry):
        for j in range(8):
            p = pk_v[k, pl.ds(j * 16, 16)]
            src_v[k, pl.ds(j * 16, 16)] = jnp.bitwise_and(p, 0xFFFF)
            dst_v[k, pl.ds(j * 16, 16)] = jnp.right_shift(p, 16)
        return carry

    lax.fori_loop(0, nchunks, chunk, 0)


def _zero_vmem(buf, rows, width):
    zv = jnp.zeros((16,), jnp.float32)

    def row(i, carry):
        for j in range(width // 16):
            buf[i, pl.ds(j * 16, 16)] = zv
        return carry

    lax.fori_loop(0, rows, row, 0)


def _fill_rows(buf, n, base):
    for j in range(n // 16):
        buf[pl.ds(j * 16, 16)] = lax.iota(jnp.int32, 16) + (base + j * 16)


def _seg_body(y_hbm, pk_hbm, agg_hbm,
              pk_v, r0, r1, si, di, row80_v, rowh_v, agg_s,
              sem, g0, g1, s0, s1):
    c = lax.axis_index("c")
    s = lax.axis_index("s")
    wid = c * _NS + s
    _fill_rows(row80_v, _KCH, wid * _KCH)
    pltpu.async_copy(pk_hbm.at[row80_v], pk_v, sem).wait()
    _zero_vmem(r0, 128, _D)
    for t in range(_RPT // 128):
        pltpu.sync_copy(r0, agg_s.at[pl.ds(s * _RPT + t * 128, 128)])
    plsc.subcore_barrier()

    bufs = (r0, r1)
    sis = (si.at[0], si.at[1])
    dis = (di.at[0], di.at[1])
    gsems = (g0, g1)
    ssems = (s0, s1)

    def gath(k, b):
        for j in range(8):
            p = pk_v[k, pl.ds(j * 16, 16)]
            si[b, pl.ds(j * 16, 16)] = jnp.bitwise_and(p, 0xFFFF)
            di[b, pl.ds(j * 16, 16)] = jnp.right_shift(p, 16)
        pltpu.async_copy(y_hbm.at[sis[b]], bufs[b], gsems[b])

    def gwait(b):
        pltpu.make_async_copy(y_hbm.at[sis[b]], bufs[b], gsems[b]).wait()

    def scat(b):
        pltpu.async_copy(bufs[b], agg_s.at[dis[b]], ssems[b], add=True)

    def swait(b):
        pltpu.make_async_copy(bufs[b], agg_s.at[dis[b]], ssems[b]).wait()

    gath(0, 0)
    gath(1, 1)

    def pair(q, carry):
        k0 = 2 * q
        gwait(0)
        scat(0)
        gwait(1)
        scat(1)

        @pl.when(k0 + 2 < _KCH)
        def _():
            swait(0)
            gath(k0 + 2, 0)
            swait(1)
            gath(k0 + 3, 1)

        return carry

    lax.fori_loop(0, _KCH // 2, pair, 0)
    swait(0)
    swait(1)
    plsc.subcore_barrier()
    for t in range(_RPT // 128):
        base = s * _RPT + t * 128
        _fill_rows(rowh_v, 128, c * _NP + base)
        pltpu.sync_copy(agg_s.at[pl.ds(base, 128)], r0)
        pltpu.sync_copy(r0, agg_hbm.at[rowh_v])


_seg = pl.kernel(
    _seg_body,
    out_type=jax.ShapeDtypeStruct((_NC * _NP, _D), jnp.float32),
    mesh=_MESH,
    compiler_params=pltpu.CompilerParams(needs_layout_passes=False),
    scratch_types=[
        pltpu.VMEM((_KCH, 128), jnp.int32),
        pltpu.VMEM((128, _D), jnp.float32),
        pltpu.VMEM((128, _D), jnp.float32),
        pltpu.VMEM((2, 128), jnp.int32),
        pltpu.VMEM((2, 128), jnp.int32),
        pltpu.VMEM((_KCH,), jnp.int32),
        pltpu.VMEM((128,), jnp.int32),
        pltpu.VMEM_SHARED((_NP, _D), jnp.float32),
        pltpu.SemaphoreType.DMA,
        pltpu.SemaphoreType.DMA,
        pltpu.SemaphoreType.DMA,
        pltpu.SemaphoreType.DMA,
        pltpu.SemaphoreType.DMA,
    ],
)


_HN = _NP // 2
_CSC = _HN + 128


def _cnt_body(pk_hbm, cnt_hbm,
              pk_v, dst_v, ones_v, blk_v, row80_v, rowh_v, cnt_s, sem):
    c = lax.axis_index("c")
    s = lax.axis_index("s")
    _fill_rows(row80_v, 80, s * 80)
    pltpu.async_copy(pk_hbm.at[row80_v], pk_v, sem).wait()
    half = c * _HN

    def chunk(kk, carry):
        for j in range(8):
            p = pk_v[kk, pl.ds(j * 16, 16)]
            for d, r in ((jnp.bitwise_and(p, 0xFFFF), 2 * kk),
                         (jnp.right_shift(p, 16), 2 * kk + 1)):
                loc = d - half
                ok = (loc >= 0) & (loc < _HN)
                dst_v[r, pl.ds(j * 16, 16)] = jnp.where(
                    ok, loc, _HN + jnp.bitwise_and(d, 127))
        return carry

    lax.fori_loop(0, 80, chunk, 0)
    _zero_vmem(blk_v, 128, _D)
    for t in range(3):
        b = s + 16 * t

        @pl.when(b < _CSC // 128)
        def _():
            pltpu.sync_copy(blk_v, cnt_s.at[pl.ds(b * 128, 128)])

    ov = jnp.where(lax.iota(jnp.int32, 16) == 0,
                   jnp.float32(1.0), jnp.float32(0.0))
    _zero_vmem(ones_v, 128, _D)

    def orow(i, carry):
        ones_v[i, pl.ds(0, 16)] = ov
        return carry

    lax.fori_loop(0, 128, orow, 0)
    plsc.subcore_barrier()

    def step(k, carry):
        pltpu.sync_copy(ones_v, cnt_s.at[dst_v.at[k]], add=True)
        return carry

    lax.fori_loop(0, 2 * _KCH, step, 0)
    plsc.subcore_barrier()
    for t in range(3):
        b = s + 16 * t

        @pl.when(b < _HN // 128)
        def _():
            pltpu.sync_copy(cnt_s.at[pl.ds(b * 128, 128)], blk_v)
            _fill_rows(rowh_v, 128, c * _HN + b * 128)
            pltpu.sync_copy(blk_v, cnt_hbm.at[rowh_v])


_cnt = pl.kernel(
    _cnt_body,
    out_type=jax.ShapeDtypeStruct((_NP, 128), jnp.float32),
    mesh=_MESH,
    compiler_params=pltpu.CompilerParams(needs_layout_passes=False),
    scratch_types=[
        pltpu.VMEM((80, 128), jnp.int32),
        pltpu.VMEM((2 * _KCH, 128), jnp.int32),
        pltpu.VMEM((128, _D), jnp.float32),
        pltpu.VMEM((128, _D), jnp.float32),
        pltpu.VMEM((80,), jnp.int32),
        pltpu.VMEM((128,), jnp.int32),
        pltpu.VMEM_SHARED((_CSC, _D), jnp.float32),
        pltpu.SemaphoreType.DMA,
    ],
)


def _dec_body(ab_hbm, pk_hbm, w2b_hbm, wa2b_hbm, bd2b_hbm,
              out_hbm,
              pk_v, src_v, dst_v, a0, b0, a1, b1, w_v, wa_v, bias_v,
              sc_v, row80_v, sem0, sem1):
    c = lax.axis_index("c")
    s = lax.axis_index("s")
    wid = c * _NS + s
    _fill_rows(row80_v, _KCH, wid * _KCH)
    pltpu.async_copy(pk_hbm.at[row80_v], pk_v, sem0).wait()
    _unpack_indices(pk_v, src_v, dst_v, _KCH)
    pltpu.sync_copy(w2b_hbm, w_v)
    pltpu.sync_copy(wa2b_hbm, wa_v)
    pltpu.sync_copy(bd2b_hbm, bias_v)
    bias = bias_v[...][0]
    ws = [w_v[pl.ds(g * 16, 16)] for g in range(4)]
    wsa = [wa_v[pl.ds(g * 16, 16)] for g in range(4)]
    lane0 = lax.iota(jnp.int32, 16) == 0

    def compute(k, av, bv):
        def edge8(e8, carry):
            for u in range(8):
                e = e8 * 8 + u
                acc = None
                for g in range(4):
                    va = av[e, pl.ds(g * 16, 16)]
                    vb = bv[e, pl.ds(_LAT + g * 16, 16)]
                    t = va + vb
                    mx = jnp.maximum(t, 0.0)
                    ex = jnp.exp(jnp.minimum(t, 0.0))
                    term = ws[g] * mx + wsa[g] * ex
                    acc = term if acc is None else acc + term
                score = jnp.sum(acc) + bias
                plsc.store_scatter(
                    sc_v, [jnp.full((16,), k, jnp.int32),
                           jnp.full((16,), e, jnp.int32)],
                    jnp.full((16,), score), mask=lane0)
            return carry

        lax.fori_loop(0, 16, edge8, 0)

    def start(k, av, bv, sm):
        pltpu.async_copy(ab_hbm.at[src_v.at[k]], av, sm)
        pltpu.async_copy(ab_hbm.at[dst_v.at[k]], bv, sm)

    def wait(k, av, bv, sm):
        pltpu.make_async_copy(ab_hbm.at[src_v.at[k]], av, sm).wait()
        pltpu.make_async_copy(ab_hbm.at[dst_v.at[k]], bv, sm).wait()

    start(0, a0, b0, sem0)

    def pair(q, carry):
        k0 = 2 * q
        start(k0 + 1, a1, b1, sem1)
        wait(k0, a0, b0, sem0)
        compute(k0, a0, b0)

        @pl.when(k0 + 2 < _KCH)
        def _():
            start(k0 + 2, a0, b0, sem0)

        wait(k0 + 1, a1, b1, sem1)
        compute(k0 + 1, a1, b1)
        return carry

    lax.fori_loop(0, _KCH // 2, pair, 0)
    pltpu.sync_copy(sc_v, out_hbm.at[wid])


_dec = pl.kernel(
    _dec_body,
    out_type=jax.ShapeDtypeStruct((_NW, _KCH, 128), jnp.float32),
    mesh=_MESH,
    compiler_params=pltpu.CompilerParams(needs_layout_passes=False),
    scratch_types=[
        pltpu.VMEM((_KCH, 128), jnp.int32),
        pltpu.VMEM((_KCH, 128), jnp.int32),
        pltpu.VMEM((_KCH, 128), jnp.int32),
        pltpu.VMEM((128, _D), jnp.float32),
        pltpu.VMEM((128, _D), jnp.float32),
        pltpu.VMEM((128, _D), jnp.float32),
        pltpu.VMEM((128, _D), jnp.float32),
        pltpu.VMEM((_LAT,), jnp.float32),
        pltpu.VMEM((_LAT,), jnp.float32),
        pltpu.VMEM((_L,), jnp.float32),
        pltpu.VMEM((_KCH, 128), jnp.float32),
        pltpu.VMEM((_KCH,), jnp.int32),
        pltpu.SemaphoreType.DMA,
        pltpu.SemaphoreType.DMA,
    ],
)


def kernel(x, edge_index, eps, W_s1, W_n1, b1, W_s2, W_n2, b2,
           Wm1, bm1, gm, btm, Wm2, bm2, Wm3, bm3,
           Wl1, bl1, gl, btl, Wl2, bl2, Wl3, bl3,
           Wd1, bd1, Wd2, bd2):
    src = edge_index[0]
    dst = edge_index[1]
    pad = _EP - _E
    srcp = jnp.concatenate([src, jnp.zeros((pad,), jnp.int32)])
    dstp = jnp.concatenate([dst, jnp.full((pad,), _N, jnp.int32)])
    packed = jnp.bitwise_or(srcp, jnp.left_shift(dstp, 16))
    pk3 = packed.reshape(_NW * _KCH, 128)
    dst2 = dstp.reshape(_NW * _KCH, 128)
    dstp2 = jnp.bitwise_or(dst2[0::2], jnp.left_shift(dst2[1::2], 16))
    dstp2 = jnp.pad(dstp2, ((0, 8), (0, 0)))

    xp = jnp.pad(x, ((0, _NP - _N), (0, 0)))
    epsp = jnp.pad(eps, ((0, _NP - _N), (0, 0)))
    w2b = Wd2[:, 0] * _SELU_SCALE
    wa2b = w2b * _SELU_ALPHA
    bd2b = jnp.broadcast_to(bd2 - jnp.sum(wa2b), (_L,))

    y1, xs = _tc1(xp, W_n1, W_s1, b1.reshape(1, -1))
    cntp = _cnt(dstp2)
    aggp1 = _seg(y1, pk3).reshape(_NC, _NP, _D)
    y2, hs = _tc2(xs, aggp1, cntp, W_n2, W_s2, b2.reshape(1, -1))
    aggp2 = _seg(y2, pk3).reshape(_NC, _NP, _D)
    ab_tab = _tc3(
        hs, aggp2, cntp, epsp,
        Wm1, bm1.reshape(1, -1), gm.reshape(1, -1), btm.reshape(1, -1),
        Wm2, bm2.reshape(1, -1), Wm3, bm3.reshape(1, -1),
        Wl1, bl1.reshape(1, -1), gl.reshape(1, -1), btl.reshape(1, -1),
        Wl2, bl2.reshape(1, -1), Wl3, bl3.reshape(1, -1),
        Wd1[:_LAT], bd1.reshape(1, -1), Wd1[_LAT:])
    scores = _dec(ab_tab, pk3, w2b, wa2b, bd2b)
    return scores.reshape(_EP)[:_E]


# speedup vs baseline: 1.0224x; 1.0224x over previous
import functools

import jax
import jax.numpy as jnp
from jax import lax
from jax.experimental import pallas as pl
from jax.experimental.pallas import tpu as pltpu
from jax.experimental.pallas import tpu_sc as plsc

_N = 10000
_E = 320000
_D = 128
_LAT = 64

_NC = 2
_NS = 16
_L = 16
_NW = _NC * _NS

_NP = 10240
_EP = 327680
_EPT = _EP // _NW
_KCH = _EPT // 128
_RPT = _NP // _NS

_SELU_ALPHA = 1.6732632423543772
_SELU_SCALE = 1.0507009873554805


def _selu(v):
    return _SELU_SCALE * jnp.where(
        v > 0, v, _SELU_ALPHA * (jnp.exp(v) - 1.0))


def _tc1_body(x_ref, wn_ref, ws_ref, b_ref, y_ref, xs_ref):
    x = x_ref[...]
    y_ref[...] = jnp.dot(x, wn_ref[...], preferred_element_type=jnp.float32)
    xs_ref[...] = (
        jnp.dot(x, ws_ref[...], preferred_element_type=jnp.float32)
        + b_ref[...])


def _tc1(xp, W_n1, W_s1, b1):
    return pl.pallas_call(
        _tc1_body,
        out_shape=(
            jax.ShapeDtypeStruct((_NP, _D), jnp.float32),
            jax.ShapeDtypeStruct((_NP, _D), jnp.float32),
        ),
    )(xp, W_n1, W_s1, b1)


def _tc2_body(xs_ref, agg_ref, cnt_ref, wn_ref, ws_ref, b_ref,
              y2_ref, hs_ref):
    agg = agg_ref[0] + agg_ref[1]
    invc = 1.0 / jnp.maximum(cnt_ref[...][:, 0:1], 1.0)
    h1 = jax.nn.relu(xs_ref[...] + agg * invc)
    y2_ref[...] = jnp.dot(h1, wn_ref[...], preferred_element_type=jnp.float32)
    hs_ref[...] = (
        jnp.dot(h1, ws_ref[...], preferred_element_type=jnp.float32)
        + b_ref[...])


def _tc2(xs, aggp, cntp, W_n2, W_s2, b2):
    R = 2048
    grid = _NP // R
    return pl.pallas_call(
        _tc2_body,
        grid=(grid,),
        in_specs=[
            pl.BlockSpec((R, _D), lambda i: (i, 0)),
            pl.BlockSpec((_NC, R, _D), lambda i: (0, i, 0)),
            pl.BlockSpec((R, 128), lambda i: (i, 0)),
            pl.BlockSpec((_D, _D), lambda i: (0, 0)),
            pl.BlockSpec((_D, _D), lambda i: (0, 0)),
            pl.BlockSpec((1, _D), lambda i: (0, 0)),
        ],
        out_specs=(
            pl.BlockSpec((R, _D), lambda i: (i, 0)),
            pl.BlockSpec((R, _D), lambda i: (i, 0)),
        ),
        out_shape=(
            jax.ShapeDtypeStruct((_NP, _D), jnp.float32),
            jax.ShapeDtypeStruct((_NP, _D), jnp.float32),
        ),
    )(xs, aggp, cntp, W_n2, W_s2, b2)


def _ln(v, g, b):
    m = jnp.mean(v, axis=-1, keepdims=True)
    var = jnp.mean((v - m) ** 2, axis=-1, keepdims=True)
    return (v - m) / jnp.sqrt(var + 1e-5) * g + b


def _tc3_body(hs_ref, agg_ref, cnt_ref, eps_ref,
              wm1_ref, bm1_ref, gm_ref, btm_ref, wm2_ref, bm2_ref,
              wm3_ref, bm3_ref,
              wl1_ref, bl1_ref, gl_ref, btl_ref, wl2_ref, bl2_ref,
              wl3_ref, bl3_ref,
              wdt_ref, bd1_ref, wdb_ref,
              ab_ref):
    agg = agg_ref[0] + agg_ref[1]
    invc = 1.0 / jnp.maximum(cnt_ref[...][:, 0:1], 1.0)
    h = hs_ref[...] + agg * invc

    m = _selu(_ln(jnp.dot(h, wm1_ref[...],
                          preferred_element_type=jnp.float32) + bm1_ref[...],
                  gm_ref[...], btm_ref[...]))
    m = _selu(jnp.dot(m, wm2_ref[...],
                      preferred_element_type=jnp.float32) + bm2_ref[...])
    mu = jnp.dot(m, wm3_ref[...],
                 preferred_element_type=jnp.float32) + bm3_ref[...]

    l = _selu(_ln(jnp.dot(h, wl1_ref[...],
                          preferred_element_type=jnp.float32) + bl1_ref[...],
                  gl_ref[...], btl_ref[...]))
    l = _selu(jnp.dot(l, wl2_ref[...],
                      preferred_element_type=jnp.float32) + bl2_ref[...])
    logstd = jnp.minimum(
        jnp.dot(l, wl3_ref[...],
                preferred_element_type=jnp.float32) + bl3_ref[...], 10.0)

    z = mu + eps_ref[...] * jnp.exp(logstd)
    a = jnp.dot(z, wdt_ref[...],
                preferred_element_type=jnp.float32) + bd1_ref[...]
    b = jnp.dot(z, wdb_ref[...],
                preferred_element_type=jnp.float32)
    ab_ref[...] = jnp.concatenate([a, b], axis=-1)


def _tc3(hs, aggp2, cntp, epsp, Wm1, bm1, gm, btm, Wm2, bm2, Wm3, bm3,
         Wl1, bl1, gl, btl, Wl2, bl2, Wl3, bl3, Wd1_top, bd1, Wd1_bot):
    R = 2048
    grid = _NP // R
    full = lambda shape: pl.BlockSpec(shape, lambda i: tuple(0 for _ in shape))
    return pl.pallas_call(
        _tc3_body,
        grid=(grid,),
        in_specs=[
            pl.BlockSpec((R, _D), lambda i: (i, 0)),
            pl.BlockSpec((_NC, R, _D), lambda i: (0, i, 0)),
            pl.BlockSpec((R, 128), lambda i: (i, 0)),
            pl.BlockSpec((R, _LAT), lambda i: (i, 0)),
            full((_D, 2 * _LAT)), full((1, 2 * _LAT)),
            full((1, 2 * _LAT)), full((1, 2 * _LAT)),
            full((2 * _LAT, 256)), full((1, 256)),
            full((256, _LAT)), full((1, _LAT)),
            full((_D, 2 * _LAT)), full((1, 2 * _LAT)),
            full((1, 2 * _LAT)), full((1, 2 * _LAT)),
            full((2 * _LAT, 256)), full((1, 256)),
            full((256, _LAT)), full((1, _LAT)),
            full((_LAT, _LAT)), full((1, _LAT)), full((_LAT, _LAT)),
        ],
        out_specs=pl.BlockSpec((R, _D), lambda i: (i, 0)),
        out_shape=jax.ShapeDtypeStruct((_NP, _D), jnp.float32),
    )(hs, aggp2, cntp, epsp, Wm1, bm1, gm, btm, Wm2, bm2, Wm3, bm3,
      Wl1, bl1, gl, btl, Wl2, bl2, Wl3, bl3, Wd1_top, bd1, Wd1_bot)


_MESH = plsc.VectorSubcoreMesh(core_axis_name="c", subcore_axis_name="s")


def _unpack_indices(pk_v, src_v, dst_v, nchunks):
    def chunk(k, carry):
        for j in range(8):
            p = pk_v[k, pl.ds(j * 16, 16)]
            src_v[k, pl.ds(j * 16, 16)] = jnp.bitwise_and(p, 0xFFFF)
            dst_v[k, pl.ds(j * 16, 16)] = jnp.right_shift(p, 16)
        return carry

    lax.fori_loop(0, nchunks, chunk, 0)


def _zero_vmem(buf, rows, width):
    zv = jnp.zeros((16,), jnp.float32)

    def row(i, carry):
        for j in range(width // 16):
            buf[i, pl.ds(j * 16, 16)] = zv
        return carry

    lax.fori_loop(0, rows, row, 0)


def _fill_rows(buf, n, base):
    for j in range(n // 16):
        buf[pl.ds(j * 16, 16)] = lax.iota(jnp.int32, 16) + (base + j * 16)


def _seg_body(y_hbm, pk_hbm, agg_hbm,
              pk_v, r0, r1, si, di, row80_v, rowh_v, agg_s,
              sem, g0, g1, s0, s1):
    c = lax.axis_index("c")
    s = lax.axis_index("s")
    wid = c * _NS + s
    _fill_rows(row80_v, _KCH, wid * _KCH)
    pltpu.async_copy(pk_hbm.at[row80_v], pk_v, sem).wait()
    _zero_vmem(r0, 128, _D)
    for t in range(_RPT // 128):
        pltpu.sync_copy(r0, agg_s.at[pl.ds(s * _RPT + t * 128, 128)])
    plsc.subcore_barrier()

    bufs = (r0, r1)
    sis = (si.at[0], si.at[1])
    dis = (di.at[0], di.at[1])
    gsems = (g0, g1)
    ssems = (s0, s1)

    def gath(k, b):
        for j in range(8):
            p = pk_v[k, pl.ds(j * 16, 16)]
            si[b, pl.ds(j * 16, 16)] = jnp.bitwise_and(p, 0xFFFF)
            di[b, pl.ds(j * 16, 16)] = jnp.right_shift(p, 16)
        pltpu.async_copy(y_hbm.at[sis[b]], bufs[b], gsems[b])

    def gwait(b):
        pltpu.make_async_copy(y_hbm.at[sis[b]], bufs[b], gsems[b]).wait()

    def scat(b):
        pltpu.async_copy(bufs[b], agg_s.at[dis[b]], ssems[b], add=True)

    def swait(b):
        pltpu.make_async_copy(bufs[b], agg_s.at[dis[b]], ssems[b]).wait()

    gath(0, 0)
    gath(1, 1)

    def pair(q, carry):
        k0 = 2 * q
        gwait(0)
        scat(0)
        gwait(1)
        scat(1)

        @pl.when(k0 + 2 < _KCH)
        def _():
            swait(0)
            gath(k0 + 2, 0)
            swait(1)
            gath(k0 + 3, 1)

        return carry

    lax.fori_loop(0, _KCH // 2, pair, 0)
    swait(0)
    swait(1)
    plsc.subcore_barrier()
    for t in range(_RPT // 128):
        base = s * _RPT + t * 128
        _fill_rows(rowh_v, 128, c * _NP + base)
        pltpu.sync_copy(agg_s.at[pl.ds(base, 128)], r0)
        pltpu.sync_copy(r0, agg_hbm.at[rowh_v])


_seg = pl.kernel(
    _seg_body,
    out_type=jax.ShapeDtypeStruct((_NC * _NP, _D), jnp.float32),
    mesh=_MESH,
    compiler_params=pltpu.CompilerParams(needs_layout_passes=False),
    scratch_types=[
        pltpu.VMEM((_KCH, 128), jnp.int32),
        pltpu.VMEM((128, _D), jnp.float32),
        pltpu.VMEM((128, _D), jnp.float32),
        pltpu.VMEM((2, 128), jnp.int32),
        pltpu.VMEM((2, 128), jnp.int32),
        pltpu.VMEM((_KCH,), jnp.int32),
        pltpu.VMEM((128,), jnp.int32),
        pltpu.VMEM_SHARED((_NP, _D), jnp.float32),
        pltpu.SemaphoreType.DMA,
        pltpu.SemaphoreType.DMA,
        pltpu.SemaphoreType.DMA,
        pltpu.SemaphoreType.DMA,
        pltpu.SemaphoreType.DMA,
    ],
)


_HN = _NP // 2
_CSC = _HN + 128


def _cnt_body(pk_hbm, cnt_hbm,
              pk_v, dst_v, ones_v, blk_v, row80_v, rowh_v, cnt_s, sem):
    c = lax.axis_index("c")
    s = lax.axis_index("s")
    _fill_rows(row80_v, 80, s * 80)
    pltpu.async_copy(pk_hbm.at[row80_v], pk_v, sem).wait()
    half = c * _HN

    def chunk(kk, carry):
        for j in range(8):
            p = pk_v[kk, pl.ds(j * 16, 16)]
            for d, r in ((jnp.bitwise_and(p, 0xFFFF), 2 * kk),
                         (jnp.right_shift(p, 16), 2 * kk + 1)):
                loc = d - half
                ok = (loc >= 0) & (loc < _HN)
                dst_v[r, pl.ds(j * 16, 16)] = jnp.where(
                    ok, loc, _HN + jnp.bitwise_and(d, 127))
        return carry

    lax.fori_loop(0, 80, chunk, 0)
    _zero_vmem(blk_v, 128, _D)
    for t in range(3):
        b = s + 16 * t

        @pl.when(b < _CSC // 128)
        def _():
            pltpu.sync_copy(blk_v, cnt_s.at[pl.ds(b * 128, 128)])

    ov = jnp.where(lax.iota(jnp.int32, 16) == 0,
                   jnp.float32(1.0), jnp.float32(0.0))
    _zero_vmem(ones_v, 128, _D)

    def orow(i, carry):
        ones_v[i, pl.ds(0, 16)] = ov
        return carry

    lax.fori_loop(0, 128, orow, 0)
    plsc.subcore_barrier()

    def step(k, carry):
        pltpu.sync_copy(ones_v, cnt_s.at[dst_v.at[k]], add=True)
        return carry

    lax.fori_loop(0, 2 * _KCH, step, 0)
    plsc.subcore_barrier()
    for t in range(3):
        b = s + 16 * t

        @pl.when(b < _HN // 128)
        def _():
            pltpu.sync_copy(cnt_s.at[pl.ds(b * 128, 128)], blk_v)
            _fill_rows(rowh_v, 128, c * _HN + b * 128)
            pltpu.sync_copy(blk_v, cnt_hbm.at[rowh_v])


_cnt = pl.kernel(
    _cnt_body,
    out_type=jax.ShapeDtypeStruct((_NP, 128), jnp.float32),
    mesh=_MESH,
    compiler_params=pltpu.CompilerParams(needs_layout_passes=False),
    scratch_types=[
        pltpu.VMEM((80, 128), jnp.int32),
        pltpu.VMEM((2 * _KCH, 128), jnp.int32),
        pltpu.VMEM((128, _D), jnp.float32),
        pltpu.VMEM((128, _D), jnp.float32),
        pltpu.VMEM((80,), jnp.int32),
        pltpu.VMEM((128,), jnp.int32),
        pltpu.VMEM_SHARED((_CSC, _D), jnp.float32),
        pltpu.SemaphoreType.DMA,
    ],
)


def _dec_body(ab_hbm, pk_hbm, w2b_hbm, wa2b_hbm, bd2b_hbm,
              out_hbm,
              pk_v, src_v, dst_v, a0, b0, a1, b1, w_v, wa_v, bias_v,
              sc_v, row80_v, st_v, sem0, sem1):
    c = lax.axis_index("c")
    s = lax.axis_index("s")
    wid = c * _NS + s
    _fill_rows(row80_v, _KCH, wid * _KCH)
    pltpu.async_copy(pk_hbm.at[row80_v], pk_v, sem0).wait()
    _unpack_indices(pk_v, src_v, dst_v, _KCH)
    pltpu.sync_copy(w2b_hbm, w_v)
    pltpu.sync_copy(wa2b_hbm, wa_v)
    pltpu.sync_copy(bd2b_hbm, bias_v)
    ws = [w_v[pl.ds(g * 16, 16)] for g in range(4)]
    wsa = [wa_v[pl.ds(g * 16, 16)] for g in range(4)]
    biasv = bias_v[...]
    i17 = lax.iota(jnp.int32, 16) * 17

    def compute(k, av, bv):
        def grp(e16, carry):
            for u in range(16):
                e = e16 * 16 + u
                acc = None
                for g in range(4):
                    va = av[e, pl.ds(g * 16, 16)]
                    vb = bv[e, pl.ds(_LAT + g * 16, 16)]
                    t = va + vb
                    mx = jnp.maximum(t, 0.0)
                    ex = jnp.exp(jnp.minimum(t, 0.0))
                    term = ws[g] * mx + wsa[g] * ex
                    acc = term if acc is None else acc + term
                st_v[pl.ds(u * 17, 16)] = acc
            red = biasv
            for col in range(16):
                red = red + plsc.load_gather(st_v, [i17 + col])
            sc_v[k, pl.ds(e16 * 16, 16)] = red
            return carry

        lax.fori_loop(0, 8, grp, 0)

    def start(k, av, bv, sm):
        pltpu.async_copy(ab_hbm.at[src_v.at[k]], av, sm)
        pltpu.async_copy(ab_hbm.at[dst_v.at[k]], bv, sm)

    def wait(k, av, bv, sm):
        pltpu.make_async_copy(ab_hbm.at[src_v.at[k]], av, sm).wait()
        pltpu.make_async_copy(ab_hbm.at[dst_v.at[k]], bv, sm).wait()

    start(0, a0, b0, sem0)

    def pair(q, carry):
        k0 = 2 * q
        start(k0 + 1, a1, b1, sem1)
        wait(k0, a0, b0, sem0)
        compute(k0, a0, b0)

        @pl.when(k0 + 2 < _KCH)
        def _():
            start(k0 + 2, a0, b0, sem0)

        wait(k0 + 1, a1, b1, sem1)
        compute(k0 + 1, a1, b1)
        return carry

    lax.fori_loop(0, _KCH // 2, pair, 0)
    pltpu.sync_copy(sc_v, out_hbm.at[wid])


_dec = pl.kernel(
    _dec_body,
    out_type=jax.ShapeDtypeStruct((_NW, _KCH, 128), jnp.float32),
    mesh=_MESH,
    compiler_params=pltpu.CompilerParams(needs_layout_passes=False),
    scratch_types=[
        pltpu.VMEM((_KCH, 128), jnp.int32),
        pltpu.VMEM((_KCH, 128), jnp.int32),
        pltpu.VMEM((_KCH, 128), jnp.int32),
        pltpu.VMEM((128, _D), jnp.float32),
        pltpu.VMEM((128, _D), jnp.float32),
        pltpu.VMEM((128, _D), jnp.float32),
        pltpu.VMEM((128, _D), jnp.float32),
        pltpu.VMEM((_LAT,), jnp.float32),
        pltpu.VMEM((_LAT,), jnp.float32),
        pltpu.VMEM((_L,), jnp.float32),
        pltpu.VMEM((_KCH, 128), jnp.float32),
        pltpu.VMEM((_KCH,), jnp.int32),
        pltpu.VMEM((272,), jnp.float32),
        pltpu.SemaphoreType.DMA,
        pltpu.SemaphoreType.DMA,
    ],
)


def kernel(x, edge_index, eps, W_s1, W_n1, b1, W_s2, W_n2, b2,
           Wm1, bm1, gm, btm, Wm2, bm2, Wm3, bm3,
           Wl1, bl1, gl, btl, Wl2, bl2, Wl3, bl3,
           Wd1, bd1, Wd2, bd2):
    src = edge_index[0]
    dst = edge_index[1]
    pad = _EP - _E
    srcp = jnp.concatenate([src, jnp.zeros((pad,), jnp.int32)])
    dstp = jnp.concatenate([dst, jnp.full((pad,), _N, jnp.int32)])
    packed = jnp.bitwise_or(srcp, jnp.left_shift(dstp, 16))
    pk3 = packed.reshape(_NW * _KCH, 128)
    dst2 = dstp.reshape(_NW * _KCH, 128)
    dstp2 = jnp.bitwise_or(dst2[0::2], jnp.left_shift(dst2[1::2], 16))
    dstp2 = jnp.pad(dstp2, ((0, 8), (0, 0)))

    xp = jnp.pad(x, ((0, _NP - _N), (0, 0)))
    epsp = jnp.pad(eps, ((0, _NP - _N), (0, 0)))
    w2b = Wd2[:, 0] * _SELU_SCALE
    wa2b = w2b * _SELU_ALPHA
    bd2b = jnp.broadcast_to(bd2 - jnp.sum(wa2b), (_L,))

    y1, xs = _tc1(xp, W_n1, W_s1, b1.reshape(1, -1))
    cntp = _cnt(dstp2)
    aggp1 = _seg(y1, pk3).reshape(_NC, _NP, _D)
    y2, hs = _tc2(xs, aggp1, cntp, W_n2, W_s2, b2.reshape(1, -1))
    aggp2 = _seg(y2, pk3).reshape(_NC, _NP, _D)
    ab_tab = _tc3(
        hs, aggp2, cntp, epsp,
        Wm1, bm1.reshape(1, -1), gm.reshape(1, -1), btm.reshape(1, -1),
        Wm2, bm2.reshape(1, -1), Wm3, bm3.reshape(1, -1),
        Wl1, bl1.reshape(1, -1), gl.reshape(1, -1), btl.reshape(1, -1),
        Wl2, bl2.reshape(1, -1), Wl3, bl3.reshape(1, -1),
        Wd1[:_LAT], bd1.reshape(1, -1), Wd1[_LAT:])
    scores = _dec(ab_tab, pk3, w2b, wa2b, bd2b)
    return scores.reshape(_EP)[:_E]


# speedup vs baseline: 1.0653x; 1.0420x over previous
import functools

import jax
import jax.numpy as jnp
from jax import lax
from jax.experimental import pallas as pl
from jax.experimental.pallas import tpu as pltpu
from jax.experimental.pallas import tpu_sc as plsc

_N = 10000
_E = 320000
_D = 128
_LAT = 64

_NC = 2
_NS = 16
_L = 16
_NW = _NC * _NS

_NP = 10240
_EP = 327680
_EPT = _EP // _NW
_KCH = _EPT // 128
_RPT = _NP // _NS
_NCH = _NW * _KCH
_KC0 = 80
_KC1 = 2 * _KCH - _KC0
_KMX = max(_KC0, _KC1)

_SELU_ALPHA = 1.6732632423543772
_SELU_SCALE = 1.0507009873554805


def _selu(v):
    return _SELU_SCALE * jnp.where(
        v > 0, v, _SELU_ALPHA * (jnp.exp(v) - 1.0))


def _tc1_body(x_ref, wn_ref, ws_ref, b_ref, y_ref, xs_ref):
    x = x_ref[...]
    y_ref[...] = jnp.dot(x, wn_ref[...], preferred_element_type=jnp.float32)
    xs_ref[...] = (
        jnp.dot(x, ws_ref[...], preferred_element_type=jnp.float32)
        + b_ref[...])


def _tc1(xp, W_n1, W_s1, b1):
    return pl.pallas_call(
        _tc1_body,
        out_shape=(
            jax.ShapeDtypeStruct((_NP, _D), jnp.float32),
            jax.ShapeDtypeStruct((_NP, _D), jnp.float32),
        ),
    )(xp, W_n1, W_s1, b1)


def _tc2_body(xs_ref, agg_ref, cnt_ref, wn_ref, ws_ref, b_ref,
              y2_ref, hs_ref):
    agg = agg_ref[0] + agg_ref[1]
    invc = 1.0 / jnp.maximum(cnt_ref[...][:, 0:1], 1.0)
    h1 = jax.nn.relu(xs_ref[...] + agg * invc)
    y2_ref[...] = jnp.dot(h1, wn_ref[...], preferred_element_type=jnp.float32)
    hs_ref[...] = (
        jnp.dot(h1, ws_ref[...], preferred_element_type=jnp.float32)
        + b_ref[...])


def _tc2(xs, aggp, cntp, W_n2, W_s2, b2):
    R = 2048
    grid = _NP // R
    return pl.pallas_call(
        _tc2_body,
        grid=(grid,),
        in_specs=[
            pl.BlockSpec((R, _D), lambda i: (i, 0)),
            pl.BlockSpec((_NC, R, _D), lambda i: (0, i, 0)),
            pl.BlockSpec((R, 128), lambda i: (i, 0)),
            pl.BlockSpec((_D, _D), lambda i: (0, 0)),
            pl.BlockSpec((_D, _D), lambda i: (0, 0)),
            pl.BlockSpec((1, _D), lambda i: (0, 0)),
        ],
        out_specs=(
            pl.BlockSpec((R, _D), lambda i: (i, 0)),
            pl.BlockSpec((R, _D), lambda i: (i, 0)),
        ),
        out_shape=(
            jax.ShapeDtypeStruct((_NP, _D), jnp.float32),
            jax.ShapeDtypeStruct((_NP, _D), jnp.float32),
        ),
    )(xs, aggp, cntp, W_n2, W_s2, b2)


def _ln(v, g, b):
    m = jnp.mean(v, axis=-1, keepdims=True)
    var = jnp.mean((v - m) ** 2, axis=-1, keepdims=True)
    return (v - m) / jnp.sqrt(var + 1e-5) * g + b


def _tc3_body(hs_ref, agg_ref, cnt_ref, eps_ref,
              wm1_ref, bm1_ref, gm_ref, btm_ref, wm2_ref, bm2_ref,
              wm3_ref, bm3_ref,
              wl1_ref, bl1_ref, gl_ref, btl_ref, wl2_ref, bl2_ref,
              wl3_ref, bl3_ref,
              wdt_ref, bd1_ref, wdb_ref,
              ab_ref):
    agg = agg_ref[0] + agg_ref[1]
    invc = 1.0 / jnp.maximum(cnt_ref[...][:, 0:1], 1.0)
    h = hs_ref[...] + agg * invc

    m = _selu(_ln(jnp.dot(h, wm1_ref[...],
                          preferred_element_type=jnp.float32) + bm1_ref[...],
                  gm_ref[...], btm_ref[...]))
    m = _selu(jnp.dot(m, wm2_ref[...],
                      preferred_element_type=jnp.float32) + bm2_ref[...])
    mu = jnp.dot(m, wm3_ref[...],
                 preferred_element_type=jnp.float32) + bm3_ref[...]

    l = _selu(_ln(jnp.dot(h, wl1_ref[...],
                          preferred_element_type=jnp.float32) + bl1_ref[...],
                  gl_ref[...], btl_ref[...]))
    l = _selu(jnp.dot(l, wl2_ref[...],
                      preferred_element_type=jnp.float32) + bl2_ref[...])
    logstd = jnp.minimum(
        jnp.dot(l, wl3_ref[...],
                preferred_element_type=jnp.float32) + bl3_ref[...], 10.0)

    z = mu + eps_ref[...] * jnp.exp(logstd)
    a = jnp.dot(z, wdt_ref[...],
                preferred_element_type=jnp.float32) + bd1_ref[...]
    b = jnp.dot(z, wdb_ref[...],
                preferred_element_type=jnp.float32)
    ab_ref[...] = jnp.concatenate([a, b], axis=-1)


def _tc3(hs, aggp2, cntp, epsp, Wm1, bm1, gm, btm, Wm2, bm2, Wm3, bm3,
         Wl1, bl1, gl, btl, Wl2, bl2, Wl3, bl3, Wd1_top, bd1, Wd1_bot):
    R = 2048
    grid = _NP // R
    full = lambda shape: pl.BlockSpec(shape, lambda i: tuple(0 for _ in shape))
    return pl.pallas_call(
        _tc3_body,
        grid=(grid,),
        in_specs=[
            pl.BlockSpec((R, _D), lambda i: (i, 0)),
            pl.BlockSpec((_NC, R, _D), lambda i: (0, i, 0)),
            pl.BlockSpec((R, 128), lambda i: (i, 0)),
            pl.BlockSpec((R, _LAT), lambda i: (i, 0)),
            full((_D, 2 * _LAT)), full((1, 2 * _LAT)),
            full((1, 2 * _LAT)), full((1, 2 * _LAT)),
            full((2 * _LAT, 256)), full((1, 256)),
            full((256, _LAT)), full((1, _LAT)),
            full((_D, 2 * _LAT)), full((1, 2 * _LAT)),
            full((1, 2 * _LAT)), full((1, 2 * _LAT)),
            full((2 * _LAT, 256)), full((1, 256)),
            full((256, _LAT)), full((1, _LAT)),
            full((_LAT, _LAT)), full((1, _LAT)), full((_LAT, _LAT)),
        ],
        out_specs=pl.BlockSpec((R, _D), lambda i: (i, 0)),
        out_shape=jax.ShapeDtypeStruct((_NP, _D), jnp.float32),
    )(hs, aggp2, cntp, epsp, Wm1, bm1, gm, btm, Wm2, bm2, Wm3, bm3,
      Wl1, bl1, gl, btl, Wl2, bl2, Wl3, bl3, Wd1_top, bd1, Wd1_bot)


_MESH = plsc.VectorSubcoreMesh(core_axis_name="c", subcore_axis_name="s")


def _unpack_indices(pk_v, src_v, dst_v, nchunks):
    def chunk(k, carry):
        for j in range(8):
            p = pk_v[k, pl.ds(j * 16, 16)]
            src_v[k, pl.ds(j * 16, 16)] = jnp.bitwise_and(p, 0xFFFF)
            dst_v[k, pl.ds(j * 16, 16)] = jnp.right_shift(p, 16)
        return carry

    lax.fori_loop(0, nchunks, chunk, 0)


def _zero_vmem(buf, rows, width):
    zv = jnp.zeros((16,), jnp.float32)

    def row(i, carry):
        for j in range(width // 16):
            buf[i, pl.ds(j * 16, 16)] = zv
        return carry

    lax.fori_loop(0, rows, row, 0)


def _fill_rows(buf, n, base):
    for j in range(n // 16):
        buf[pl.ds(j * 16, 16)] = lax.iota(jnp.int32, 16) + (base + j * 16)


def _seg_body(y_hbm, pk_hbm, agg_hbm,
              pk_v, r0, r1, si, di, rowm_v, rowh_v, agg_s, sem, g0, g1,
              s0, s1):
    c = lax.axis_index("c")
    s = lax.axis_index("s")
    _zero_vmem(r0, 128, _D)
    for t in range(_RPT // 128):
        pltpu.sync_copy(r0, agg_s.at[pl.ds(s * _RPT + t * 128, 128)])
    plsc.subcore_barrier()

    bufs = (r0, r1)
    sis = (si.at[0], si.at[1])
    dis = (di.at[0], di.at[1])
    gsems = (g0, g1)
    ssems = (s0, s1)

    def gath(k, b):
        for j in range(8):
            p = pk_v[k, pl.ds(j * 16, 16)]
            si[b, pl.ds(j * 16, 16)] = jnp.bitwise_and(p, 0xFFFF)
            di[b, pl.ds(j * 16, 16)] = jnp.right_shift(p, 16)
        pltpu.async_copy(y_hbm.at[sis[b]], bufs[b], gsems[b])

    def gwait(b):
        pltpu.make_async_copy(y_hbm.at[sis[b]], bufs[b], gsems[b]).wait()

    def scat(b):
        pltpu.async_copy(bufs[b], agg_s.at[dis[b]], ssems[b], add=True)

    def swait(b):
        pltpu.make_async_copy(bufs[b], agg_s.at[dis[b]], ssems[b]).wait()

    def run(kc, base):
        _fill_rows(rowm_v, _KMX, base)
        pltpu.async_copy(pk_hbm.at[rowm_v], pk_v, sem).wait()
        gath(0, 0)
        gath(1, 1)

        def pair(q, carry):
            k0 = 2 * q
            gwait(0)
            scat(0)
            gwait(1)
            scat(1)

            @pl.when(k0 + 2 < kc)
            def _():
                swait(0)
                gath(k0 + 2, 0)
                swait(1)
                gath(k0 + 3, 1)

            return carry

        lax.fori_loop(0, kc // 2, pair, 0)
        swait(0)
        swait(1)

    run(_KCH, (c * _NS + s) * _KCH)

    plsc.subcore_barrier()
    for t in range(_RPT // 128):
        base = s * _RPT + t * 128
        _fill_rows(rowh_v, 128, c * _NP + base)
        pltpu.sync_copy(agg_s.at[pl.ds(base, 128)], r0)
        pltpu.sync_copy(r0, agg_hbm.at[rowh_v])


_seg = pl.kernel(
    _seg_body,
    out_type=jax.ShapeDtypeStruct((_NC * _NP, _D), jnp.float32),
    mesh=_MESH,
    compiler_params=pltpu.CompilerParams(needs_layout_passes=False),
    scratch_types=[
        pltpu.VMEM((_KMX, 128), jnp.int32),
        pltpu.VMEM((128, _D), jnp.float32),
        pltpu.VMEM((128, _D), jnp.float32),
        pltpu.VMEM((2, 128), jnp.int32),
        pltpu.VMEM((2, 128), jnp.int32),
        pltpu.VMEM((_KMX,), jnp.int32),
        pltpu.VMEM((128,), jnp.int32),
        pltpu.VMEM_SHARED((_NP, _D), jnp.float32),
        pltpu.SemaphoreType.DMA,
        pltpu.SemaphoreType.DMA,
        pltpu.SemaphoreType.DMA,
        pltpu.SemaphoreType.DMA,
        pltpu.SemaphoreType.DMA,
    ],
)


_HN = _NP // 2
_CSC = _HN + 128


def _cnt_body(pk_hbm, cnt_hbm,
              pk_v, dst_v, ones_v, blk_v, row80_v, rowh_v, cnt_s, sem):
    c = lax.axis_index("c")
    s = lax.axis_index("s")
    _fill_rows(row80_v, 80, s * 80)
    pltpu.async_copy(pk_hbm.at[row80_v], pk_v, sem).wait()
    half = c * _HN

    def chunk(kk, carry):
        for j in range(8):
            p = pk_v[kk, pl.ds(j * 16, 16)]
            for d, r in ((jnp.bitwise_and(p, 0xFFFF), 2 * kk),
                         (jnp.right_shift(p, 16), 2 * kk + 1)):
                loc = d - half
                ok = (loc >= 0) & (loc < _HN)
                dst_v[r, pl.ds(j * 16, 16)] = jnp.where(
                    ok, loc, _HN + jnp.bitwise_and(d, 127))
        return carry

    lax.fori_loop(0, 80, chunk, 0)
    _zero_vmem(blk_v, 128, _D)
    for t in range(3):
        b = s + 16 * t

        @pl.when(b < _CSC // 128)
        def _():
            pltpu.sync_copy(blk_v, cnt_s.at[pl.ds(b * 128, 128)])

    ov = jnp.where(lax.iota(jnp.int32, 16) == 0,
                   jnp.float32(1.0), jnp.float32(0.0))
    _zero_vmem(ones_v, 128, _D)

    def orow(i, carry):
        ones_v[i, pl.ds(0, 16)] = ov
        return carry

    lax.fori_loop(0, 128, orow, 0)
    plsc.subcore_barrier()

    def step(k, carry):
        pltpu.sync_copy(ones_v, cnt_s.at[dst_v.at[k]], add=True)
        return carry

    lax.fori_loop(0, 2 * _KCH, step, 0)
    plsc.subcore_barrier()
    for t in range(3):
        b = s + 16 * t

        @pl.when(b < _HN // 128)
        def _():
            pltpu.sync_copy(cnt_s.at[pl.ds(b * 128, 128)], blk_v)
            _fill_rows(rowh_v, 128, c * _HN + b * 128)
            pltpu.sync_copy(blk_v, cnt_hbm.at[rowh_v])


_cnt = pl.kernel(
    _cnt_body,
    out_type=jax.ShapeDtypeStruct((_NP, 128), jnp.float32),
    mesh=_MESH,
    compiler_params=pltpu.CompilerParams(needs_layout_passes=False),
    scratch_types=[
        pltpu.VMEM((80, 128), jnp.int32),
        pltpu.VMEM((2 * _KCH, 128), jnp.int32),
        pltpu.VMEM((128, _D), jnp.float32),
        pltpu.VMEM((128, _D), jnp.float32),
        pltpu.VMEM((80,), jnp.int32),
        pltpu.VMEM((128,), jnp.int32),
        pltpu.VMEM_SHARED((_CSC, _D), jnp.float32),
        pltpu.SemaphoreType.DMA,
    ],
)


def _dec_body(ab_hbm, pk_hbm, w2b_hbm, wa2b_hbm, bd2b_hbm,
              out_hbm,
              pk_v, a0, b0, a1, b1, si, di, w_v, wa_v, bias_v,
              sc_v, rowm_v, st_v, sem0, sem1):
    c = lax.axis_index("c")
    s = lax.axis_index("s")
    pltpu.sync_copy(w2b_hbm, w_v)
    pltpu.sync_copy(wa2b_hbm, wa_v)
    pltpu.sync_copy(bd2b_hbm, bias_v)
    ws = [w_v[pl.ds(g * 16, 16)] for g in range(4)]
    wsa = [wa_v[pl.ds(g * 16, 16)] for g in range(4)]
    biasv = bias_v[...]
    i17 = lax.iota(jnp.int32, 16) * 17
    sis = (si.at[0], si.at[1])
    dis = (di.at[0], di.at[1])

    def compute(k, av, bv):
        def grp(e16, carry):
            for u in range(16):
                e = e16 * 16 + u
                acc = None
                for g in range(4):
                    va = av[e, pl.ds(g * 16, 16)]
                    vb = bv[e, pl.ds(_LAT + g * 16, 16)]
                    t = va + vb
                    mx = jnp.maximum(t, 0.0)
                    ex = jnp.exp(jnp.minimum(t, 0.0))
                    term = ws[g] * mx + wsa[g] * ex
                    acc = term if acc is None else acc + term
                st_v[pl.ds(u * 17, 16)] = acc
            red = biasv
            for col in range(16):
                red = red + plsc.load_gather(st_v, [i17 + col])
            sc_v[k, pl.ds(e16 * 16, 16)] = red
            return carry

        lax.fori_loop(0, 8, grp, 0)

    def start(k, av, bv, b, sm):
        for j in range(8):
            p = pk_v[k, pl.ds(j * 16, 16)]
            si[b, pl.ds(j * 16, 16)] = jnp.bitwise_and(p, 0xFFFF)
            di[b, pl.ds(j * 16, 16)] = jnp.right_shift(p, 16)
        pltpu.async_copy(ab_hbm.at[sis[b]], av, sm)
        pltpu.async_copy(ab_hbm.at[dis[b]], bv, sm)

    def wait(av, bv, b, sm):
        pltpu.make_async_copy(ab_hbm.at[sis[b]], av, sm).wait()
        pltpu.make_async_copy(ab_hbm.at[dis[b]], bv, sm).wait()

    def run(kc, base):
        _fill_rows(rowm_v, _KMX, base)
        pltpu.async_copy(pk_hbm.at[rowm_v], pk_v, sem0).wait()
        start(0, a0, b0, 0, sem0)

        def pair(q, carry):
            k0 = 2 * q
            start(k0 + 1, a1, b1, 1, sem1)
            wait(a0, b0, 0, sem0)
            compute(k0, a0, b0)

            @pl.when(k0 + 2 < kc)
            def _():
                start(k0 + 2, a0, b0, 0, sem0)

            wait(a1, b1, 1, sem1)
            compute(k0 + 1, a1, b1)
            return carry

        lax.fori_loop(0, kc // 2, pair, 0)
        pltpu.sync_copy(sc_v.at[pl.ds(0, kc)], out_hbm.at[pl.ds(base, kc)])

    run(_KCH, (c * _NS + s) * _KCH)


_dec = pl.kernel(
    _dec_body,
    out_type=jax.ShapeDtypeStruct((_NCH, 128), jnp.float32),
    mesh=_MESH,
    compiler_params=pltpu.CompilerParams(needs_layout_passes=False),
    scratch_types=[
        pltpu.VMEM((_KMX, 128), jnp.int32),
        pltpu.VMEM((128, _D), jnp.float32),
        pltpu.VMEM((128, _D), jnp.float32),
        pltpu.VMEM((128, _D), jnp.float32),
        pltpu.VMEM((128, _D), jnp.float32),
        pltpu.VMEM((2, 128), jnp.int32),
        pltpu.VMEM((2, 128), jnp.int32),
        pltpu.VMEM((_LAT,), jnp.float32),
        pltpu.VMEM((_LAT,), jnp.float32),
        pltpu.VMEM((_L,), jnp.float32),
        pltpu.VMEM((_KMX, 128), jnp.float32),
        pltpu.VMEM((_KMX,), jnp.int32),
        pltpu.VMEM((272,), jnp.float32),
        pltpu.SemaphoreType.DMA,
        pltpu.SemaphoreType.DMA,
    ],
)


def kernel(x, edge_index, eps, W_s1, W_n1, b1, W_s2, W_n2, b2,
           Wm1, bm1, gm, btm, Wm2, bm2, Wm3, bm3,
           Wl1, bl1, gl, btl, Wl2, bl2, Wl3, bl3,
           Wd1, bd1, Wd2, bd2):
    src = edge_index[0]
    dst = edge_index[1]
    pad = _EP - _E
    srcp = jnp.concatenate([src, jnp.zeros((pad,), jnp.int32)])
    dstp = jnp.concatenate([dst, jnp.full((pad,), _N, jnp.int32)])
    packed = jnp.bitwise_or(srcp, jnp.left_shift(dstp, 16))
    pk3 = jnp.pad(packed.reshape(_NCH, 128), ((0, _KMX), (0, 0)))
    dst2 = dstp.reshape(_NW * _KCH, 128)
    dstp2 = jnp.bitwise_or(dst2[0::2], jnp.left_shift(dst2[1::2], 16))
    dstp2 = jnp.pad(dstp2, ((0, 8), (0, 0)))

    xp = jnp.pad(x, ((0, _NP - _N), (0, 0)))
    epsp = jnp.pad(eps, ((0, _NP - _N), (0, 0)))
    w2b = Wd2[:, 0] * _SELU_SCALE
    wa2b = w2b * _SELU_ALPHA
    bd2b = jnp.broadcast_to(bd2 - jnp.sum(wa2b), (_L,))

    y1, xs = _tc1(xp, W_n1, W_s1, b1.reshape(1, -1))
    cntp = _cnt(dstp2)
    aggp1 = _seg(y1, pk3).reshape(_NC, _NP, _D)
    y2, hs = _tc2(xs, aggp1, cntp, W_n2, W_s2, b2.reshape(1, -1))
    aggp2 = _seg(y2, pk3).reshape(_NC, _NP, _D)
    ab_tab = _tc3(
        hs, aggp2, cntp, epsp,
        Wm1, bm1.reshape(1, -1), gm.reshape(1, -1), btm.reshape(1, -1),
        Wm2, bm2.reshape(1, -1), Wm3, bm3.reshape(1, -1),
        Wl1, bl1.reshape(1, -1), gl.reshape(1, -1), btl.reshape(1, -1),
        Wl2, bl2.reshape(1, -1), Wl3, bl3.reshape(1, -1),
        Wd1[:_LAT], bd1.reshape(1, -1), Wd1[_LAT:])
    scores = _dec(ab_tab, pk3, w2b, wa2b, bd2b)
    return scores.reshape(_EP)[:_E]
